# baseline (device time: 16423 ns/iter reference)
import jax
import jax.numpy as jnp
from jax import lax
from jax.experimental import pallas as pl
from jax.experimental.pallas import tpu as pltpu

N_DEV = 4
VC = 512


def kernel(x, W, labels):
    t, d = x.shape
    _, v_local = W.shape
    nk = v_local // VC

    def body(x_ref, w_ref, labels_ref, out_ref,
             s_acc, lab_acc, comm_ref, send_sems, recv_sems):
        k = pl.program_id(0)
        my_pos = lax.axis_index("i")
        barrier_sem = pltpu.get_barrier_semaphore()

        @pl.when(k == 0)
        def _():
            for dist in range(1, N_DEV):
                peer = lax.rem(my_pos + dist, N_DEV)
                pl.semaphore_signal(
                    barrier_sem, inc=1,
                    device_id=(peer,), device_id_type=pl.DeviceIdType.MESH,
                )
            s_acc[:] = jnp.zeros((t,), jnp.float32)
            lab_acc[:] = jnp.zeros((t,), jnp.float32)

        logits = jnp.dot(x_ref[:, :], w_ref[:, :],
                         preferred_element_type=jnp.float32)
        s_acc[:] += jnp.sum(jnp.exp(logits), axis=1)
        loc = labels_ref[:] - my_pos * v_local - k * VC
        cols = lax.broadcasted_iota(jnp.int32, (t, VC), 1)
        lab_acc[:] += jnp.sum(jnp.where(cols == loc[:, None], logits, 0.0),
                              axis=1)

        @pl.when(k == nk - 1)
        def _():
            comm_ref[0, 0, :] = s_acc[:]
            comm_ref[0, 1, :] = lab_acc[:]
            pl.semaphore_wait(barrier_sem, N_DEV - 1)
            rdmas = []
            for dist in range(1, N_DEV):
                peer = lax.rem(my_pos + dist, N_DEV)
                rdma = pltpu.make_async_remote_copy(
                    src_ref=comm_ref.at[0],
                    dst_ref=comm_ref.at[dist],
                    send_sem=send_sems.at[dist - 1],
                    recv_sem=recv_sems.at[dist - 1],
                    device_id=(peer,),
                    device_id_type=pl.DeviceIdType.MESH,
                )
                rdma.start()
                rdmas.append(rdma)
            for rdma in rdmas:
                rdma.wait()
            S = jnp.sum(comm_ref[:, 0, :], axis=0)
            L = jnp.sum(comm_ref[:, 1, :], axis=0)
            out_ref[:] = jnp.log(S) - L

    return pl.pallas_call(
        body,
        grid=(nk,),
        out_shape=jax.ShapeDtypeStruct((t,), jnp.float32),
        in_specs=[
            pl.BlockSpec((t, d), lambda k: (0, 0)),
            pl.BlockSpec((d, VC), lambda k: (0, k)),
            pl.BlockSpec((t,), lambda k: (0,)),
        ],
        out_specs=pl.BlockSpec((t,), lambda k: (0,)),
        scratch_shapes=[
            pltpu.VMEM((t,), jnp.float32),
            pltpu.VMEM((t,), jnp.float32),
            pltpu.VMEM((N_DEV, 2, t), jnp.float32),
            pltpu.SemaphoreType.DMA((N_DEV - 1,)),
            pltpu.SemaphoreType.DMA((N_DEV - 1,)),
        ],
        compiler_params=pltpu.CompilerParams(collective_id=0),
    )(x, W, labels)


# device time: 12331 ns/iter; 1.3318x vs baseline; 1.3318x over previous
import jax
import jax.numpy as jnp
from jax import lax
from jax.experimental import pallas as pl
from jax.experimental.pallas import tpu as pltpu

N_DEV = 4


def kernel(x, W, labels):
    t, d = x.shape
    _, v_local = W.shape

    def body(x_ref, w_ref, labels_ref, out_ref, comm_ref, send_sems, recv_sems):
        my_pos = lax.axis_index("i")

        barrier_sem = pltpu.get_barrier_semaphore()
        for dist in range(1, N_DEV):
            peer = lax.rem(my_pos + dist, N_DEV)
            pl.semaphore_signal(
                barrier_sem, inc=1,
                device_id=(peer,), device_id_type=pl.DeviceIdType.MESH,
            )

        logits = jnp.dot(x_ref[:, :], w_ref[:, :],
                         preferred_element_type=jnp.float32)
        s = jnp.sum(jnp.exp(logits), axis=1)

        loc = labels_ref[:] - my_pos * v_local
        cols = lax.broadcasted_iota(jnp.int32, (t, v_local), 1)
        lab = jnp.sum(jnp.where(cols == loc[:, None], logits, 0.0), axis=1)

        comm_ref[0, 0, :] = s
        comm_ref[0, 1, :] = lab

        pl.semaphore_wait(barrier_sem, N_DEV - 1)

        rdmas = []
        for dist in range(1, N_DEV):
            peer = lax.rem(my_pos + dist, N_DEV)
            rdma = pltpu.make_async_remote_copy(
                src_ref=comm_ref.at[0],
                dst_ref=comm_ref.at[dist],
                send_sem=send_sems.at[dist - 1],
                recv_sem=recv_sems.at[dist - 1],
                device_id=(peer,),
                device_id_type=pl.DeviceIdType.MESH,
            )
            rdma.start()
            rdmas.append(rdma)
        for rdma in rdmas:
            rdma.wait()

        S = jnp.sum(comm_ref[:, 0, :], axis=0)
        L = jnp.sum(comm_ref[:, 1, :], axis=0)
        out_ref[:] = jnp.log(S) - L

    return pl.pallas_call(
        body,
        out_shape=jax.ShapeDtypeStruct((t,), jnp.float32),
        in_specs=[
            pl.BlockSpec(memory_space=pltpu.VMEM),
            pl.BlockSpec(memory_space=pltpu.VMEM),
            pl.BlockSpec(memory_space=pltpu.VMEM),
        ],
        out_specs=pl.BlockSpec(memory_space=pltpu.VMEM),
        scratch_shapes=[
            pltpu.VMEM((N_DEV, 2, t), jnp.float32),
            pltpu.SemaphoreType.DMA((N_DEV - 1,)),
            pltpu.SemaphoreType.DMA((N_DEV - 1,)),
        ],
        compiler_params=pltpu.CompilerParams(collective_id=0),
    )(x, W, labels)
